# Initial kernel scaffold; baseline (speedup 1.0000x reference)
import jax
import jax.numpy as jnp
from jax.experimental import pallas as pl


def kernel(input_batch, embeddings):
    raise NotImplementedError("write your pallas kernel here")


# speedup vs baseline: 5.5313x; 5.5313x over previous
import functools

import jax
import jax.numpy as jnp
from jax import lax
from jax.experimental import pallas as pl
from jax.experimental.pallas import tpu as pltpu
from jax.experimental.pallas import tpu_sc as plsc

_NC = 2
_NS = 16
_NW = _NC * _NS


def _make_sc_gather(total, emb, chunk):
    per_w = total // _NW
    nchunk = per_w // chunk
    mesh = plsc.VectorSubcoreMesh(core_axis_name="c", subcore_axis_name="s")

    @functools.partial(
        pl.kernel,
        mesh=mesh,
        out_type=jax.ShapeDtypeStruct((total, emb), jnp.float32),
        scratch_types=[
            pltpu.VMEM((chunk,), jnp.int32),
            pltpu.VMEM((chunk, emb), jnp.float32),
            pltpu.SemaphoreType.DMA,
        ],
        compiler_params=pltpu.CompilerParams(use_tc_tiling_on_sc=False),
    )
    def k(idx_hbm, table_hbm, out_hbm, idx_v, rows_v, sem):
        wid = lax.axis_index("s") * _NC + lax.axis_index("c")
        base = wid * per_w

        def body(i, _):
            off = base + i * chunk
            pltpu.sync_copy(idx_hbm.at[pl.ds(off, chunk)], idx_v)
            pltpu.async_copy(table_hbm.at[idx_v], rows_v, sem).wait()
            pltpu.sync_copy(rows_v, out_hbm.at[pl.ds(off, chunk)])
            return ()

        lax.fori_loop(0, nchunk, body, ())

    return k


def kernel(input_batch, embeddings):
    seq, batch = input_batch.shape
    vocab, emb = embeddings.shape
    total = seq * batch
    idx = input_batch.reshape(total).astype(jnp.int32)
    out = _make_sc_gather(total, emb, chunk=1600)(idx, embeddings)
    return out.reshape(seq, batch, emb)


# speedup vs baseline: 5.6453x; 1.0206x over previous
import functools

import jax
import jax.numpy as jnp
from jax import lax
from jax.experimental import pallas as pl
from jax.experimental.pallas import tpu as pltpu
from jax.experimental.pallas import tpu_sc as plsc

_NC = 2
_NS = 16
_NW = _NC * _NS


def _make_sc_gather(total, emb, chunk):
    per_w = total // _NW
    nchunk = per_w // chunk
    assert nchunk % 2 == 0 and nchunk * chunk == per_w
    nstep = nchunk // 2
    mesh = plsc.VectorSubcoreMesh(core_axis_name="c", subcore_axis_name="s")

    @functools.partial(
        pl.kernel,
        mesh=mesh,
        out_type=jax.ShapeDtypeStruct((total, emb), jnp.float32),
        scratch_types=[
            pltpu.VMEM((chunk,), jnp.int32),
            pltpu.VMEM((chunk,), jnp.int32),
            pltpu.VMEM((chunk, emb), jnp.float32),
            pltpu.VMEM((chunk, emb), jnp.float32),
            pltpu.SemaphoreType.DMA,
            pltpu.SemaphoreType.DMA,
        ],
        compiler_params=pltpu.CompilerParams(use_tc_tiling_on_sc=False),
    )
    def k(idx_hbm, table_hbm, out_hbm, idx_v0, idx_v1, rows_v0, rows_v1,
          sem0, sem1):
        wid = lax.axis_index("s") * _NC + lax.axis_index("c")
        base = wid * per_w

        pltpu.sync_copy(idx_hbm.at[pl.ds(base, chunk)], idx_v0)
        pltpu.async_copy(table_hbm.at[idx_v0], rows_v0, sem0)

        def body(s, _):
            i1 = 2 * s + 1
            off1 = base + i1 * chunk
            pltpu.sync_copy(idx_hbm.at[pl.ds(off1, chunk)], idx_v1)
            pltpu.async_copy(table_hbm.at[idx_v1], rows_v1, sem1)

            pltpu.make_async_copy(table_hbm.at[idx_v0], rows_v0, sem0).wait()
            off0 = base + 2 * s * chunk
            pltpu.sync_copy(rows_v0, out_hbm.at[pl.ds(off0, chunk)])

            @pl.when(s < nstep - 1)
            def _():
                off2 = base + (2 * s + 2) * chunk
                pltpu.sync_copy(idx_hbm.at[pl.ds(off2, chunk)], idx_v0)
                pltpu.async_copy(table_hbm.at[idx_v0], rows_v0, sem0)

            pltpu.make_async_copy(table_hbm.at[idx_v1], rows_v1, sem1).wait()
            pltpu.sync_copy(rows_v1, out_hbm.at[pl.ds(off1, chunk)])
            return ()

        lax.fori_loop(0, nstep, body, ())

    return k


def kernel(input_batch, embeddings):
    seq, batch = input_batch.shape
    vocab, emb = embeddings.shape
    total = seq * batch
    idx = input_batch.reshape(total).astype(jnp.int32)
    out = _make_sc_gather(total, emb, chunk=1600)(idx, embeddings)
    return out.reshape(seq, batch, emb)
